# baseline (speedup 1.0000x reference)
import functools

import jax
import jax.numpy as jnp
from jax import lax
from jax.experimental import pallas as pl
from jax.experimental.pallas import tpu as pltpu
from jax.experimental.pallas import tpu_sc as plsc

B = 16384
NEG = 20
D = 64
L = 16
NC = 2
NS = 16
NW = NC * NS
BPW = B // NW
CHUNK = 32
NCHUNK = BPW // CHUNK
NGROW = 128
NGIDX = CHUNK * NEG // NGROW
PACK_BR = 8000


def _sc_dots(center_h, outside_h, neg_h, wc2_h, wo2_h, uo_out, uk_out,
             idx_c, idx_o, idx_n, scb, sob, snb, c_rows, o_rows, n_rows,
             uo_buf, uk_buf, sem):
    wid = lax.axis_index("s") * NC + lax.axis_index("c")
    base = wid * BPW
    pltpu.sync_copy(center_h.at[pl.ds(base, BPW)], idx_c.at[pl.ds(0, BPW)])
    pltpu.sync_copy(outside_h.at[pl.ds(base, BPW)], idx_o.at[pl.ds(0, BPW)])
    pltpu.sync_copy(neg_h.at[pl.ds(base * NEG, BPW * NEG)], idx_n)

    def chunk_body(t, _):
        for v in range(CHUNK // L):
            scb[pl.ds(v * L, L)] = idx_c[pl.ds(t * CHUNK + v * L, L)] >> 1
            sob[pl.ds(v * L, L)] = idx_o[pl.ds(t * CHUNK + v * L, L)] >> 1
        for v in range(CHUNK * NEG // L):
            snb[pl.ds(v * L, L)] = (
                idx_n[pl.ds(t * CHUNK * NEG + v * L, L)] >> 1)
        cps = [pltpu.async_copy(wc2_h.at[scb], c_rows, sem),
               pltpu.async_copy(wo2_h.at[sob], o_rows, sem)]
        for j in range(NGIDX):
            cps.append(
                pltpu.async_copy(wo2_h.at[snb.at[pl.ds(j * NGROW, NGROW)]],
                                 n_rows.at[pl.ds(j * NGROW, NGROW)], sem))
        for cp in cps:
            cp.wait()

        def item_body(m, _):
            gi = t * CHUNK + m
            offc = (idx_c[pl.ds(gi, L)][0] & 1) * D
            offo = (idx_o[pl.ds(gi, L)][0] & 1) * D
            nv0 = (idx_n[pl.ds(gi * NEG, L)] & 1) * D
            nv1 = (idx_n[pl.ds(gi * NEG + NEG - L, L)] & 1) * D
            cj = [c_rows[m, pl.ds(offc + j * L, L)] for j in range(D // L)]
            oj = [o_rows[m, pl.ds(offo + j * L, L)] for j in range(D // L)]
            p = cj[0] * oj[0]
            for j in range(1, D // L):
                p = p + cj[j] * oj[j]
            sj = [n_rows[m * NEG, pl.ds(nv0[0] + j * L, L)]
                  for j in range(D // L)]
            for k in range(1, NEG):
                offn = nv0[k] if k < L else nv1[k - (NEG - L)]
                for j in range(D // L):
                    sj[j] = sj[j] + n_rows[m * NEG + k,
                                           pl.ds(offn + j * L, L)]
            q = cj[0] * sj[0]
            for j in range(1, D // L):
                q = q + cj[j] * sj[j]
            uo_buf[pl.ds(gi * L, L)] = p
            uk_buf[pl.ds(gi * L, L)] = -q
            return 0

        lax.fori_loop(0, CHUNK, item_body, 0)
        return 0

    lax.fori_loop(0, NCHUNK, chunk_body, 0)
    pltpu.sync_copy(uo_buf, uo_out.at[pl.ds(base * L, BPW * L)])
    pltpu.sync_copy(uk_buf, uk_out.at[pl.ds(base * L, BPW * L)])


@functools.partial(
    pl.kernel,
    mesh=plsc.VectorSubcoreMesh(core_axis_name="c", subcore_axis_name="s"),
    out_type=[jax.ShapeDtypeStruct((B * L,), jnp.float32),
              jax.ShapeDtypeStruct((B * L,), jnp.float32)],
    scratch_types=[
        pltpu.VMEM((BPW + L,), jnp.int32),
        pltpu.VMEM((BPW + L,), jnp.int32),
        pltpu.VMEM((BPW * NEG,), jnp.int32),
        pltpu.VMEM((CHUNK,), jnp.int32),
        pltpu.VMEM((CHUNK,), jnp.int32),
        pltpu.VMEM((CHUNK * NEG,), jnp.int32),
        pltpu.VMEM((CHUNK, 2 * D), jnp.float32),
        pltpu.VMEM((CHUNK, 2 * D), jnp.float32),
        pltpu.VMEM((CHUNK * NEG, 2 * D), jnp.float32),
        pltpu.VMEM((BPW * L,), jnp.float32),
        pltpu.VMEM((BPW * L,), jnp.float32),
        pltpu.SemaphoreType.DMA,
    ],
)
def _sc_kernel(center_h, outside_h, neg_h, wc2_h, wo2_h, uo_out, uk_out,
               idx_c, idx_o, idx_n, scb, sob, snb, c_rows, o_rows, n_rows,
               uo_buf, uk_buf, sem):
    _sc_dots(center_h, outside_h, neg_h, wc2_h, wo2_h, uo_out, uk_out,
             idx_c, idx_o, idx_n, scb, sob, snb, c_rows, o_rows, n_rows,
             uo_buf, uk_buf, sem)


def _pack_body(in_ref, out_ref):
    x = in_ref[...].reshape(in_ref.shape[0] // 2, 2, D)
    out_ref[...] = jnp.concatenate([x[:, 0, :], x[:, 1, :]], axis=1)


def _pack_pairs(w):
    v = w.shape[0]
    return pl.pallas_call(
        _pack_body,
        grid=(v // PACK_BR,),
        in_specs=[pl.BlockSpec((PACK_BR, D), lambda i: (i, 0))],
        out_specs=pl.BlockSpec((PACK_BR // 2, 2 * D), lambda i: (i, 0)),
        out_shape=jax.ShapeDtypeStruct((v // 2, 2 * D), jnp.float32),
    )(w)


def _loss_body(uo_ref, uk_ref, out_ref):
    a = uo_ref[...].reshape(B // 8, 8, L).sum(axis=-1)
    b = uk_ref[...].reshape(B // 8, 8, L).sum(axis=-1)

    def logsig(x):
        return jnp.minimum(x, 0.0) - jnp.log(1.0 + jnp.exp(-jnp.abs(x)))

    out_ref[...] = jnp.full((1, 1), -jnp.sum(logsig(a) + logsig(b)) / B)


def kernel(center, outside, negative, W_center, W_outside):
    center = center.reshape(B)
    outside = outside.reshape(B)
    neg = negative.reshape(B * NEG)
    wc2 = _pack_pairs(W_center)
    wo2 = _pack_pairs(W_outside)
    uo, uk = _sc_kernel(center, outside, neg, wc2, wo2)
    loss = pl.pallas_call(
        _loss_body,
        out_shape=jax.ShapeDtypeStruct((1, 1), jnp.float32),
    )(uo.reshape(B // 8, 8 * L), uk.reshape(B // 8, 8 * L))
    return loss[0, 0]


# speedup vs baseline: 1.5781x; 1.5781x over previous
import functools

import jax
import jax.numpy as jnp
from jax import lax
from jax.experimental import pallas as pl
from jax.experimental.pallas import tpu as pltpu
from jax.experimental.pallas import tpu_sc as plsc

B = 16384
NEG = 20
D = 64
L = 16
NC = 2
NS = 16
NW = NC * NS
BPW = B // NW
CHUNK = 32
NCHUNK = BPW // CHUNK
NGROW = 128
NGIDX = CHUNK * NEG // NGROW
PACK_BR = 4096


def _sc_dots(center_h, outside_h, neg_h, wc2_h, wo2_h, uo_out, uk_out,
             idx_c, idx_o, idx_n, scb, sob, snb, c_rows, o_rows, n_rows,
             uo_buf, uk_buf, sem):
    wid = lax.axis_index("s") * NC + lax.axis_index("c")
    base = wid * BPW
    pltpu.sync_copy(center_h.at[pl.ds(base, BPW)], idx_c.at[pl.ds(0, BPW)])
    pltpu.sync_copy(outside_h.at[pl.ds(base, BPW)], idx_o.at[pl.ds(0, BPW)])
    pltpu.sync_copy(neg_h.at[pl.ds(base * NEG, BPW * NEG)], idx_n)

    def chunk_body(t, _):
        for v in range(CHUNK // L):
            scb[pl.ds(v * L, L)] = idx_c[pl.ds(t * CHUNK + v * L, L)] >> 1
            sob[pl.ds(v * L, L)] = idx_o[pl.ds(t * CHUNK + v * L, L)] >> 1
        for v in range(CHUNK * NEG // L):
            snb[pl.ds(v * L, L)] = (
                idx_n[pl.ds(t * CHUNK * NEG + v * L, L)] >> 1)
        cps = [pltpu.async_copy(wc2_h.at[scb], c_rows, sem),
               pltpu.async_copy(wo2_h.at[sob], o_rows, sem)]
        for j in range(NGIDX):
            cps.append(
                pltpu.async_copy(wo2_h.at[snb.at[pl.ds(j * NGROW, NGROW)]],
                                 n_rows.at[pl.ds(j * NGROW, NGROW)], sem))
        for cp in cps:
            cp.wait()

        def item_body(m, _):
            gi = t * CHUNK + m
            offc = (idx_c[pl.ds(gi, L)][0] & 1) * D
            offo = (idx_o[pl.ds(gi, L)][0] & 1) * D
            nv0 = (idx_n[pl.ds(gi * NEG, L)] & 1) * D
            nv1 = (idx_n[pl.ds(gi * NEG + NEG - L, L)] & 1) * D
            cj = [c_rows[m, pl.ds(offc + j * L, L)] for j in range(D // L)]
            oj = [o_rows[m, pl.ds(offo + j * L, L)] for j in range(D // L)]
            p = cj[0] * oj[0]
            for j in range(1, D // L):
                p = p + cj[j] * oj[j]
            sj = [n_rows[m * NEG, pl.ds(nv0[0] + j * L, L)]
                  for j in range(D // L)]
            for k in range(1, NEG):
                offn = nv0[k] if k < L else nv1[k - (NEG - L)]
                for j in range(D // L):
                    sj[j] = sj[j] + n_rows[m * NEG + k,
                                           pl.ds(offn + j * L, L)]
            q = cj[0] * sj[0]
            for j in range(1, D // L):
                q = q + cj[j] * sj[j]
            uo_buf[pl.ds(gi * L, L)] = p
            uk_buf[pl.ds(gi * L, L)] = -q
            return 0

        lax.fori_loop(0, CHUNK, item_body, 0)
        return 0

    lax.fori_loop(0, NCHUNK, chunk_body, 0)
    pltpu.sync_copy(uo_buf, uo_out.at[pl.ds(base * L, BPW * L)])
    pltpu.sync_copy(uk_buf, uk_out.at[pl.ds(base * L, BPW * L)])


@functools.partial(
    pl.kernel,
    mesh=plsc.VectorSubcoreMesh(core_axis_name="c", subcore_axis_name="s"),
    out_type=[jax.ShapeDtypeStruct((B * L,), jnp.float32),
              jax.ShapeDtypeStruct((B * L,), jnp.float32)],
    scratch_types=[
        pltpu.VMEM((BPW + L,), jnp.int32),
        pltpu.VMEM((BPW + L,), jnp.int32),
        pltpu.VMEM((BPW * NEG,), jnp.int32),
        pltpu.VMEM((CHUNK,), jnp.int32),
        pltpu.VMEM((CHUNK,), jnp.int32),
        pltpu.VMEM((CHUNK * NEG,), jnp.int32),
        pltpu.VMEM((CHUNK, 2 * D), jnp.float32),
        pltpu.VMEM((CHUNK, 2 * D), jnp.float32),
        pltpu.VMEM((CHUNK * NEG, 2 * D), jnp.float32),
        pltpu.VMEM((BPW * L,), jnp.float32),
        pltpu.VMEM((BPW * L,), jnp.float32),
        pltpu.SemaphoreType.DMA,
    ],
)
def _sc_kernel(center_h, outside_h, neg_h, wc2_h, wo2_h, uo_out, uk_out,
               idx_c, idx_o, idx_n, scb, sob, snb, c_rows, o_rows, n_rows,
               uo_buf, uk_buf, sem):
    _sc_dots(center_h, outside_h, neg_h, wc2_h, wo2_h, uo_out, uk_out,
             idx_c, idx_o, idx_n, scb, sob, snb, c_rows, o_rows, n_rows,
             uo_buf, uk_buf, sem)


def _pack_body(in_ref, out_ref):
    x = in_ref[...].T.reshape(PACK_BR // 2, 2, D)
    out_ref[...] = jnp.concatenate([x[:, 0, :], x[:, 1, :]], axis=1)


def _pack_pairs(w):
    v = w.shape[0]
    return pl.pallas_call(
        _pack_body,
        grid=((v + PACK_BR - 1) // PACK_BR,),
        in_specs=[pl.BlockSpec((D, PACK_BR), lambda i: (0, i))],
        out_specs=pl.BlockSpec((PACK_BR // 2, 2 * D), lambda i: (i, 0)),
        out_shape=jax.ShapeDtypeStruct((v // 2, 2 * D), jnp.float32),
    )(w.T)


def _loss_body(uo_ref, uk_ref, out_ref):
    a = uo_ref[...].reshape(B // 8, 8, L).sum(axis=-1)
    b = uk_ref[...].reshape(B // 8, 8, L).sum(axis=-1)

    def logsig(x):
        return jnp.minimum(x, 0.0) - jnp.log(1.0 + jnp.exp(-jnp.abs(x)))

    out_ref[...] = jnp.full((1, 1), -jnp.sum(logsig(a) + logsig(b)) / B)


def kernel(center, outside, negative, W_center, W_outside):
    center = center.reshape(B)
    outside = outside.reshape(B)
    neg = negative.reshape(B * NEG)
    wc2 = _pack_pairs(W_center)
    wo2 = _pack_pairs(W_outside)
    uo, uk = _sc_kernel(center, outside, neg, wc2, wo2)
    loss = pl.pallas_call(
        _loss_body,
        out_shape=jax.ShapeDtypeStruct((1, 1), jnp.float32),
    )(uo.reshape(B // 8, 8 * L), uk.reshape(B // 8, 8 * L))
    return loss[0, 0]


# speedup vs baseline: 1.9151x; 1.2136x over previous
import functools

import jax
import jax.numpy as jnp
from jax import lax
from jax.experimental import pallas as pl
from jax.experimental.pallas import tpu as pltpu
from jax.experimental.pallas import tpu_sc as plsc

B = 16384
NEG = 20
D = 64
L = 16
NC = 2
NS = 16
NW = NC * NS
BPW = B // NW
CHUNK = 32
NCHUNK = BPW // CHUNK
NGROW = 128
NGIDX = CHUNK * NEG // NGROW
PACK_BR = 4096


def _sc_dots(center_h, outside_h, neg_h, wcp_h, wop_h, uo_out, uk_out,
             idx_c, idx_o, idx_n, c_rows, o_rows, n_rows,
             uo_buf, uk_buf, sem):
    wid = lax.axis_index("s") * NC + lax.axis_index("c")
    base = wid * BPW
    pltpu.sync_copy(center_h.at[pl.ds(base, BPW)], idx_c)
    pltpu.sync_copy(outside_h.at[pl.ds(base, BPW)], idx_o)
    pltpu.sync_copy(neg_h.at[pl.ds(base * NEG, BPW * NEG)], idx_n)

    def chunk_body(t, _):
        cps = [pltpu.async_copy(wcp_h.at[idx_c.at[pl.ds(t * CHUNK, CHUNK)]],
                                c_rows, sem),
               pltpu.async_copy(wop_h.at[idx_o.at[pl.ds(t * CHUNK, CHUNK)]],
                                o_rows, sem)]
        for j in range(NGIDX):
            cps.append(pltpu.async_copy(
                wop_h.at[idx_n.at[pl.ds(t * CHUNK * NEG + j * NGROW, NGROW)]],
                n_rows.at[pl.ds(j * NGROW, NGROW)], sem))
        for cp in cps:
            cp.wait()

        def item_body(m, _):
            gi = t * CHUNK + m
            cj = [c_rows[m, pl.ds(j * L, L)] for j in range(D // L)]
            p = cj[0] * o_rows[m, pl.ds(0, L)]
            for j in range(1, D // L):
                p = p + cj[j] * o_rows[m, pl.ds(j * L, L)]
            sj = [n_rows[m * NEG, pl.ds(j * L, L)] for j in range(D // L)]
            for k in range(1, NEG):
                for j in range(D // L):
                    sj[j] = sj[j] + n_rows[m * NEG + k, pl.ds(j * L, L)]
            q = cj[0] * sj[0]
            for j in range(1, D // L):
                q = q + cj[j] * sj[j]
            uo_buf[pl.ds(gi * L, L)] = p
            uk_buf[pl.ds(gi * L, L)] = -q
            return 0

        lax.fori_loop(0, CHUNK, item_body, 0)
        return 0

    lax.fori_loop(0, NCHUNK, chunk_body, 0)
    pltpu.sync_copy(uo_buf, uo_out.at[pl.ds(base * L, BPW * L)])
    pltpu.sync_copy(uk_buf, uk_out.at[pl.ds(base * L, BPW * L)])


@functools.partial(
    pl.kernel,
    mesh=plsc.VectorSubcoreMesh(core_axis_name="c", subcore_axis_name="s"),
    out_type=[jax.ShapeDtypeStruct((B * L,), jnp.float32),
              jax.ShapeDtypeStruct((B * L,), jnp.float32)],
    scratch_types=[
        pltpu.VMEM((BPW,), jnp.int32),
        pltpu.VMEM((BPW,), jnp.int32),
        pltpu.VMEM((BPW * NEG,), jnp.int32),
        pltpu.VMEM((CHUNK, 2 * D), jnp.float32),
        pltpu.VMEM((CHUNK, 2 * D), jnp.float32),
        pltpu.VMEM((CHUNK * NEG, 2 * D), jnp.float32),
        pltpu.VMEM((BPW * L,), jnp.float32),
        pltpu.VMEM((BPW * L,), jnp.float32),
        pltpu.SemaphoreType.DMA,
    ],
)
def _sc_kernel(center_h, outside_h, neg_h, wcp_h, wop_h, uo_out, uk_out,
               idx_c, idx_o, idx_n, c_rows, o_rows, n_rows,
               uo_buf, uk_buf, sem):
    _sc_dots(center_h, outside_h, neg_h, wcp_h, wop_h, uo_out, uk_out,
             idx_c, idx_o, idx_n, c_rows, o_rows, n_rows,
             uo_buf, uk_buf, sem)


def _pad_body(in_ref, out_ref):
    out_ref[:, pl.ds(0, D)] = in_ref[...].T


def _transpose_pad(w):
    v = w.shape[0]
    return pl.pallas_call(
        _pad_body,
        grid=((v + PACK_BR - 1) // PACK_BR,),
        in_specs=[pl.BlockSpec((D, PACK_BR), lambda i: (0, i))],
        out_specs=pl.BlockSpec((PACK_BR, 2 * D), lambda i: (i, 0)),
        out_shape=jax.ShapeDtypeStruct((v, 2 * D), jnp.float32),
    )(w.T)


def _loss_body(uo_ref, uk_ref, out_ref):
    a = uo_ref[...].reshape(B // 8, 8, L).sum(axis=-1)
    b = uk_ref[...].reshape(B // 8, 8, L).sum(axis=-1)

    def logsig(x):
        return jnp.minimum(x, 0.0) - jnp.log(1.0 + jnp.exp(-jnp.abs(x)))

    out_ref[...] = jnp.full((1, 1), -jnp.sum(logsig(a) + logsig(b)) / B)


def kernel(center, outside, negative, W_center, W_outside):
    center = center.reshape(B)
    outside = outside.reshape(B)
    neg = negative.reshape(B * NEG)
    wcp = _transpose_pad(W_center)
    wop = _transpose_pad(W_outside)
    uo, uk = _sc_kernel(center, outside, neg, wcp, wop)
    loss = pl.pallas_call(
        _loss_body,
        out_shape=jax.ShapeDtypeStruct((1, 1), jnp.float32),
    )(uo.reshape(B // 8, 8 * L), uk.reshape(B // 8, 8 * L))
    return loss[0, 0]


# speedup vs baseline: 2.3223x; 1.2126x over previous
import functools

import jax
import jax.numpy as jnp
from jax import lax
from jax.experimental import pallas as pl
from jax.experimental.pallas import tpu as pltpu
from jax.experimental.pallas import tpu_sc as plsc

B = 16384
NEG = 20
D = 64
L = 16
NC = 2
NS = 16
NW = NC * NS
BPW = B // NW
CHUNK = 32
NCHUNK = BPW // CHUNK
NGROW = 128
NGIDX = CHUNK * NEG // NGROW
PACK_BR = 4096
NBLK = 125
S = NBLK * PACK_BR


def _sc_dots(center_h, outside_h, neg_h, wcp_h, wop_h, uo_out, uk_out,
             idx_c, idx_o, idx_n, scb, sob, snb, c_rows, o_rows, n_rows,
             uo_buf, uk_buf, sem):
    wid = lax.axis_index("s") * NC + lax.axis_index("c")
    base = wid * BPW
    pltpu.sync_copy(center_h.at[pl.ds(base, BPW)], idx_c.at[pl.ds(0, BPW)])
    pltpu.sync_copy(outside_h.at[pl.ds(base, BPW)], idx_o.at[pl.ds(0, BPW)])
    pltpu.sync_copy(neg_h.at[pl.ds(base * NEG, BPW * NEG)], idx_n)

    def chunk_body(t, _):
        for v in range(CHUNK // L):
            iv = idx_c[pl.ds(t * CHUNK + v * L, L)]
            scb[pl.ds(v * L, L)] = jnp.where(iv >= S, iv - S, iv)
            ov = idx_o[pl.ds(t * CHUNK + v * L, L)]
            sob[pl.ds(v * L, L)] = jnp.where(ov >= S, ov - S, ov)
        for v in range(CHUNK * NEG // L):
            nv = idx_n[pl.ds(t * CHUNK * NEG + v * L, L)]
            snb[pl.ds(v * L, L)] = jnp.where(nv >= S, nv - S, nv)
        cps = [pltpu.async_copy(wcp_h.at[scb], c_rows, sem),
               pltpu.async_copy(wop_h.at[sob], o_rows, sem)]
        for j in range(NGIDX):
            cps.append(pltpu.async_copy(
                wop_h.at[snb.at[pl.ds(j * NGROW, NGROW)]],
                n_rows.at[pl.ds(j * NGROW, NGROW)], sem))
        for cp in cps:
            cp.wait()

        def item_body(m, _):
            gi = t * CHUNK + m
            offc = jnp.where(idx_c[pl.ds(gi, L)][0] >= S, D, 0)
            offo = jnp.where(idx_o[pl.ds(gi, L)][0] >= S, D, 0)
            nv0 = jnp.where(idx_n[pl.ds(gi * NEG, L)] >= S, D, 0)
            nv1 = jnp.where(idx_n[pl.ds(gi * NEG + NEG - L, L)] >= S, D, 0)
            cj = [c_rows[m, pl.ds(offc + j * L, L)] for j in range(D // L)]
            oj = [o_rows[m, pl.ds(offo + j * L, L)] for j in range(D // L)]
            p = cj[0] * oj[0]
            for j in range(1, D // L):
                p = p + cj[j] * oj[j]
            sj = [n_rows[m * NEG, pl.ds(nv0[0] + j * L, L)]
                  for j in range(D // L)]
            for k in range(1, NEG):
                offn = nv0[k] if k < L else nv1[k - (NEG - L)]
                for j in range(D // L):
                    sj[j] = sj[j] + n_rows[m * NEG + k,
                                           pl.ds(offn + j * L, L)]
            q = cj[0] * sj[0]
            for j in range(1, D // L):
                q = q + cj[j] * sj[j]
            uo_buf[pl.ds(gi * L, L)] = p
            uk_buf[pl.ds(gi * L, L)] = -q
            return 0

        lax.fori_loop(0, CHUNK, item_body, 0)
        return 0

    lax.fori_loop(0, NCHUNK, chunk_body, 0)
    pltpu.sync_copy(uo_buf, uo_out.at[pl.ds(base * L, BPW * L)])
    pltpu.sync_copy(uk_buf, uk_out.at[pl.ds(base * L, BPW * L)])


@functools.partial(
    pl.kernel,
    mesh=plsc.VectorSubcoreMesh(core_axis_name="c", subcore_axis_name="s"),
    out_type=[jax.ShapeDtypeStruct((B * L,), jnp.float32),
              jax.ShapeDtypeStruct((B * L,), jnp.float32)],
    scratch_types=[
        pltpu.VMEM((BPW + L,), jnp.int32),
        pltpu.VMEM((BPW + L,), jnp.int32),
        pltpu.VMEM((BPW * NEG,), jnp.int32),
        pltpu.VMEM((CHUNK,), jnp.int32),
        pltpu.VMEM((CHUNK,), jnp.int32),
        pltpu.VMEM((CHUNK * NEG,), jnp.int32),
        pltpu.VMEM((CHUNK, 2 * D), jnp.float32),
        pltpu.VMEM((CHUNK, 2 * D), jnp.float32),
        pltpu.VMEM((CHUNK * NEG, 2 * D), jnp.float32),
        pltpu.VMEM((BPW * L,), jnp.float32),
        pltpu.VMEM((BPW * L,), jnp.float32),
        pltpu.SemaphoreType.DMA,
    ],
)
def _sc_kernel(center_h, outside_h, neg_h, wcp_h, wop_h, uo_out, uk_out,
               idx_c, idx_o, idx_n, scb, sob, snb, c_rows, o_rows, n_rows,
               uo_buf, uk_buf, sem):
    _sc_dots(center_h, outside_h, neg_h, wcp_h, wop_h, uo_out, uk_out,
             idx_c, idx_o, idx_n, scb, sob, snb, c_rows, o_rows, n_rows,
             uo_buf, uk_buf, sem)


def _pack_body(in1_ref, in2_ref, out_ref):
    out_ref[:, pl.ds(0, D)] = in1_ref[...].T
    out_ref[:, pl.ds(D, D)] = in2_ref[...].T


def _transpose_split(w):
    return pl.pallas_call(
        _pack_body,
        grid=(NBLK,),
        in_specs=[pl.BlockSpec((D, PACK_BR), lambda i: (0, i)),
                  pl.BlockSpec((D, PACK_BR),
                               lambda i: (0, jnp.minimum(i + NBLK,
                                                         NBLK * 2 - 6)))],
        out_specs=pl.BlockSpec((PACK_BR, 2 * D), lambda i: (i, 0)),
        out_shape=jax.ShapeDtypeStruct((S, 2 * D), jnp.float32),
    )(w.T, w.T)


def _loss_body(uo_ref, uk_ref, out_ref):
    a = uo_ref[...].reshape(B // 8, 8, L).sum(axis=-1)
    b = uk_ref[...].reshape(B // 8, 8, L).sum(axis=-1)

    def logsig(x):
        return jnp.minimum(x, 0.0) - jnp.log(1.0 + jnp.exp(-jnp.abs(x)))

    out_ref[...] = jnp.full((1, 1), -jnp.sum(logsig(a) + logsig(b)) / B)


def kernel(center, outside, negative, W_center, W_outside):
    center = center.reshape(B)
    outside = outside.reshape(B)
    neg = negative.reshape(B * NEG)
    wcp = _transpose_split(W_center)
    wop = _transpose_split(W_outside)
    uo, uk = _sc_kernel(center, outside, neg, wcp, wop)
    loss = pl.pallas_call(
        _loss_body,
        out_shape=jax.ShapeDtypeStruct((1, 1), jnp.float32),
    )(uo.reshape(B // 8, 8 * L), uk.reshape(B // 8, 8 * L))
    return loss[0, 0]


# speedup vs baseline: 2.5917x; 1.1160x over previous
import functools

import jax
import jax.numpy as jnp
from jax import lax
from jax.experimental import pallas as pl
from jax.experimental.pallas import tpu as pltpu
from jax.experimental.pallas import tpu_sc as plsc

B = 16384
NEG = 20
D = 64
L = 16
NC = 2
NS = 16
NW = NC * NS
BPW = B // NW
CHUNK = 32
NCHUNK = BPW // CHUNK
NGROW = 128
NGIDX = CHUNK * NEG // NGROW
PACK_BR = 8192
NBLK = 62
LASTBLK = (1000000 + PACK_BR - 1) // PACK_BR - 1
S = NBLK * PACK_BR


def _sc_dots(center_h, outside_h, neg_h, wcp_h, wop_h, uo_out, uk_out,
             idx_c, idx_o, idx_n, scb, sob, snb, c_rows, o_rows, n_rows,
             uo_buf, uk_buf, sem):
    wid = lax.axis_index("s") * NC + lax.axis_index("c")
    base = wid * BPW
    pltpu.sync_copy(center_h.at[pl.ds(base, BPW)], idx_c.at[pl.ds(0, BPW)])
    pltpu.sync_copy(outside_h.at[pl.ds(base, BPW)], idx_o.at[pl.ds(0, BPW)])
    pltpu.sync_copy(neg_h.at[pl.ds(base * NEG, BPW * NEG)], idx_n)

    def chunk_body(t, _):
        for v in range(CHUNK // L):
            iv = idx_c[pl.ds(t * CHUNK + v * L, L)]
            scb[pl.ds(v * L, L)] = jnp.where(iv >= S, iv - S, iv)
            ov = idx_o[pl.ds(t * CHUNK + v * L, L)]
            sob[pl.ds(v * L, L)] = jnp.where(ov >= S, ov - S, ov)
        for v in range(CHUNK * NEG // L):
            nv = idx_n[pl.ds(t * CHUNK * NEG + v * L, L)]
            snb[pl.ds(v * L, L)] = jnp.where(nv >= S, nv - S, nv)
        cps = [pltpu.async_copy(wcp_h.at[scb], c_rows, sem),
               pltpu.async_copy(wop_h.at[sob], o_rows, sem)]
        for j in range(NGIDX):
            cps.append(pltpu.async_copy(
                wop_h.at[snb.at[pl.ds(j * NGROW, NGROW)]],
                n_rows.at[pl.ds(j * NGROW, NGROW)], sem))
        for cp in cps:
            cp.wait()

        def item_body(m, _):
            gi = t * CHUNK + m
            offc = jnp.where(idx_c[pl.ds(gi, L)][0] >= S, D, 0)
            offo = jnp.where(idx_o[pl.ds(gi, L)][0] >= S, D, 0)
            nv0 = jnp.where(idx_n[pl.ds(gi * NEG, L)] >= S, D, 0)
            nv1 = jnp.where(idx_n[pl.ds(gi * NEG + NEG - L, L)] >= S, D, 0)
            cj = [c_rows[m, pl.ds(offc + j * L, L)] for j in range(D // L)]
            oj = [o_rows[m, pl.ds(offo + j * L, L)] for j in range(D // L)]
            p = cj[0] * oj[0]
            for j in range(1, D // L):
                p = p + cj[j] * oj[j]
            sj = [n_rows[m * NEG, pl.ds(nv0[0] + j * L, L)]
                  for j in range(D // L)]
            for k in range(1, NEG):
                offn = nv0[k] if k < L else nv1[k - (NEG - L)]
                for j in range(D // L):
                    sj[j] = sj[j] + n_rows[m * NEG + k,
                                           pl.ds(offn + j * L, L)]
            q = cj[0] * sj[0]
            for j in range(1, D // L):
                q = q + cj[j] * sj[j]
            uo_buf[pl.ds(gi * L, L)] = p
            uk_buf[pl.ds(gi * L, L)] = -q
            return 0

        lax.fori_loop(0, CHUNK, item_body, 0)
        return 0

    lax.fori_loop(0, NCHUNK, chunk_body, 0)
    pltpu.sync_copy(uo_buf, uo_out.at[pl.ds(base * L, BPW * L)])
    pltpu.sync_copy(uk_buf, uk_out.at[pl.ds(base * L, BPW * L)])


@functools.partial(
    pl.kernel,
    mesh=plsc.VectorSubcoreMesh(core_axis_name="c", subcore_axis_name="s"),
    out_type=[jax.ShapeDtypeStruct((B * L,), jnp.float32),
              jax.ShapeDtypeStruct((B * L,), jnp.float32)],
    scratch_types=[
        pltpu.VMEM((BPW + L,), jnp.int32),
        pltpu.VMEM((BPW + L,), jnp.int32),
        pltpu.VMEM((BPW * NEG,), jnp.int32),
        pltpu.VMEM((CHUNK,), jnp.int32),
        pltpu.VMEM((CHUNK,), jnp.int32),
        pltpu.VMEM((CHUNK * NEG,), jnp.int32),
        pltpu.VMEM((CHUNK, 2 * D), jnp.float32),
        pltpu.VMEM((CHUNK, 2 * D), jnp.float32),
        pltpu.VMEM((CHUNK * NEG, 2 * D), jnp.float32),
        pltpu.VMEM((BPW * L,), jnp.float32),
        pltpu.VMEM((BPW * L,), jnp.float32),
        pltpu.SemaphoreType.DMA,
    ],
)
def _sc_kernel(center_h, outside_h, neg_h, wcp_h, wop_h, uo_out, uk_out,
               idx_c, idx_o, idx_n, scb, sob, snb, c_rows, o_rows, n_rows,
               uo_buf, uk_buf, sem):
    _sc_dots(center_h, outside_h, neg_h, wcp_h, wop_h, uo_out, uk_out,
             idx_c, idx_o, idx_n, scb, sob, snb, c_rows, o_rows, n_rows,
             uo_buf, uk_buf, sem)


def _pack_body(in1_ref, in2_ref, out_ref):
    out_ref[:, pl.ds(0, D)] = in1_ref[...].T
    out_ref[:, pl.ds(D, D)] = in2_ref[...].T


def _transpose_split(w):
    return pl.pallas_call(
        _pack_body,
        grid=(NBLK,),
        in_specs=[pl.BlockSpec((D, PACK_BR), lambda i: (0, i)),
                  pl.BlockSpec((D, PACK_BR),
                               lambda i: (0, jnp.minimum(i + NBLK,
                                                         LASTBLK)))],
        out_specs=pl.BlockSpec((PACK_BR, 2 * D), lambda i: (i, 0)),
        out_shape=jax.ShapeDtypeStruct((S, 2 * D), jnp.float32),
    )(w.T, w.T)


def _loss_body(uo_ref, uk_ref, out_ref):
    a = uo_ref[...].reshape(B // 8, 8, L).sum(axis=-1)
    b = uk_ref[...].reshape(B // 8, 8, L).sum(axis=-1)

    def logsig(x):
        return jnp.minimum(x, 0.0) - jnp.log(1.0 + jnp.exp(-jnp.abs(x)))

    out_ref[...] = jnp.full((1, 1), -jnp.sum(logsig(a) + logsig(b)) / B)


def kernel(center, outside, negative, W_center, W_outside):
    center = center.reshape(B)
    outside = outside.reshape(B)
    neg = negative.reshape(B * NEG)
    wcp = _transpose_split(W_center)
    wop = _transpose_split(W_outside)
    uo, uk = _sc_kernel(center, outside, neg, wcp, wop)
    loss = pl.pallas_call(
        _loss_body,
        out_shape=jax.ShapeDtypeStruct((1, 1), jnp.float32),
    )(uo.reshape(B // 8, 8 * L), uk.reshape(B // 8, 8 * L))
    return loss[0, 0]


# speedup vs baseline: 2.9075x; 1.1218x over previous
import functools

import jax
import jax.numpy as jnp
from jax import lax
from jax.experimental import pallas as pl
from jax.experimental.pallas import tpu as pltpu
from jax.experimental.pallas import tpu_sc as plsc

B = 16384
NEG = 20
D = 64
L = 16
NC = 2
NS = 16
NW = NC * NS
BPW = B // NW

CH_A = 16
NCH_A = BPW // CH_A
NGROW = 80
NGIDX = CH_A * NEG // NGROW

CH_C = 64
NCH_C = BPW // CH_C

PACK_BR = 8192
NBLK = 62
LASTBLK = (1000000 + PACK_BR - 1) // PACK_BR - 1
S = NBLK * PACK_BR


def _sc_a(outside_h, neg_h, wop_h, o_out, s_out,
          idx_o, idx_n, sob, snb, o_rows, n_rows, o_buf, s_buf, sem):
    wid = lax.axis_index("s") * NC + lax.axis_index("c")
    base = wid * BPW
    pltpu.sync_copy(outside_h.at[pl.ds(base, BPW)], idx_o.at[pl.ds(0, BPW)])
    pltpu.sync_copy(neg_h.at[pl.ds(base * NEG, BPW * NEG)], idx_n)

    def chunk_body(t, _):
        for v in range(CH_A // L):
            ov = idx_o[pl.ds(t * CH_A + v * L, L)]
            sob[pl.ds(v * L, L)] = jnp.where(ov >= S, ov - S, ov)
        for v in range(CH_A * NEG // L):
            nv = idx_n[pl.ds(t * CH_A * NEG + v * L, L)]
            snb[pl.ds(v * L, L)] = jnp.where(nv >= S, nv - S, nv)
        cps = [pltpu.async_copy(wop_h.at[sob], o_rows, sem)]
        for j in range(NGIDX):
            cps.append(pltpu.async_copy(
                wop_h.at[snb.at[pl.ds(j * NGROW, NGROW)]],
                n_rows.at[pl.ds(j * NGROW, NGROW)], sem))
        for cp in cps:
            cp.wait()

        def item_body(m, _):
            gi = t * CH_A + m
            offo = jnp.where(idx_o[pl.ds(gi, L)][0] >= S, D, 0)
            nv0 = jnp.where(idx_n[pl.ds(gi * NEG, L)] >= S, D, 0)
            nv1 = jnp.where(idx_n[pl.ds(gi * NEG + NEG - L, L)] >= S, D, 0)
            sj = [n_rows[m * NEG, pl.ds(nv0[0] + j * L, L)]
                  for j in range(D // L)]
            for k in range(1, NEG):
                offn = nv0[k] if k < L else nv1[k - (NEG - L)]
                for j in range(D // L):
                    sj[j] = sj[j] + n_rows[m * NEG + k,
                                           pl.ds(offn + j * L, L)]
            for j in range(D // L):
                o_buf[pl.ds(gi * D + j * L, L)] = o_rows[m, pl.ds(offo
                                                                  + j * L, L)]
                s_buf[pl.ds(gi * D + j * L, L)] = sj[j]
            return 0

        lax.fori_loop(0, CH_A, item_body, 0)
        return 0

    lax.fori_loop(0, NCH_A, chunk_body, 0)
    pltpu.sync_copy(o_buf, o_out.at[pl.ds(base * D, BPW * D)])
    pltpu.sync_copy(s_buf, s_out.at[pl.ds(base * D, BPW * D)])


@functools.partial(
    pl.kernel,
    mesh=plsc.VectorSubcoreMesh(core_axis_name="c", subcore_axis_name="s"),
    out_type=[jax.ShapeDtypeStruct((B * D,), jnp.float32),
              jax.ShapeDtypeStruct((B * D,), jnp.float32)],
    scratch_types=[
        pltpu.VMEM((BPW + L,), jnp.int32),
        pltpu.VMEM((BPW * NEG,), jnp.int32),
        pltpu.VMEM((CH_A,), jnp.int32),
        pltpu.VMEM((CH_A * NEG,), jnp.int32),
        pltpu.VMEM((CH_A, 2 * D), jnp.float32),
        pltpu.VMEM((CH_A * NEG, 2 * D), jnp.float32),
        pltpu.VMEM((BPW * D,), jnp.float32),
        pltpu.VMEM((BPW * D,), jnp.float32),
        pltpu.SemaphoreType.DMA,
    ],
)
def _sc_kernel_a(outside_h, neg_h, wop_h, o_out, s_out,
                 idx_o, idx_n, sob, snb, o_rows, n_rows, o_buf, s_buf, sem):
    _sc_a(outside_h, neg_h, wop_h, o_out, s_out,
          idx_o, idx_n, sob, snb, o_rows, n_rows, o_buf, s_buf, sem)


def _sc_c(center_h, o_h, s_h, wcp_h, uo_out, uk_out,
          idx_c, scb, c_rows, o_l, s_l, uo_buf, uk_buf, sem):
    wid = lax.axis_index("s") * NC + lax.axis_index("c")
    base = wid * BPW
    pltpu.sync_copy(center_h.at[pl.ds(base, BPW)], idx_c.at[pl.ds(0, BPW)])

    def chunk_body(t, _):
        for v in range(CH_C // L):
            iv = idx_c[pl.ds(t * CH_C + v * L, L)]
            scb[pl.ds(v * L, L)] = jnp.where(iv >= S, iv - S, iv)
        cp = pltpu.async_copy(wcp_h.at[scb], c_rows, sem)
        pltpu.sync_copy(o_h.at[pl.ds((base + t * CH_C) * D, CH_C * D)], o_l)
        pltpu.sync_copy(s_h.at[pl.ds((base + t * CH_C) * D, CH_C * D)], s_l)
        cp.wait()

        def item_body(m, _):
            gi = t * CH_C + m
            offc = jnp.where(idx_c[pl.ds(gi, L)][0] >= S, D, 0)
            cj = [c_rows[m, pl.ds(offc + j * L, L)] for j in range(D // L)]
            p = cj[0] * o_l[pl.ds(m * D, L)]
            q = cj[0] * s_l[pl.ds(m * D, L)]
            for j in range(1, D // L):
                p = p + cj[j] * o_l[pl.ds(m * D + j * L, L)]
                q = q + cj[j] * s_l[pl.ds(m * D + j * L, L)]
            uo_buf[pl.ds(gi * L, L)] = p
            uk_buf[pl.ds(gi * L, L)] = -q
            return 0

        lax.fori_loop(0, CH_C, item_body, 0)
        return 0

    lax.fori_loop(0, NCH_C, chunk_body, 0)
    pltpu.sync_copy(uo_buf, uo_out.at[pl.ds(base * L, BPW * L)])
    pltpu.sync_copy(uk_buf, uk_out.at[pl.ds(base * L, BPW * L)])


@functools.partial(
    pl.kernel,
    mesh=plsc.VectorSubcoreMesh(core_axis_name="c", subcore_axis_name="s"),
    out_type=[jax.ShapeDtypeStruct((B * L,), jnp.float32),
              jax.ShapeDtypeStruct((B * L,), jnp.float32)],
    scratch_types=[
        pltpu.VMEM((BPW + L,), jnp.int32),
        pltpu.VMEM((CH_C,), jnp.int32),
        pltpu.VMEM((CH_C, 2 * D), jnp.float32),
        pltpu.VMEM((CH_C * D,), jnp.float32),
        pltpu.VMEM((CH_C * D,), jnp.float32),
        pltpu.VMEM((BPW * L,), jnp.float32),
        pltpu.VMEM((BPW * L,), jnp.float32),
        pltpu.SemaphoreType.DMA,
    ],
)
def _sc_kernel_c(center_h, o_h, s_h, wcp_h, uo_out, uk_out,
                 idx_c, scb, c_rows, o_l, s_l, uo_buf, uk_buf, sem):
    _sc_c(center_h, o_h, s_h, wcp_h, uo_out, uk_out,
          idx_c, scb, c_rows, o_l, s_l, uo_buf, uk_buf, sem)


def _pack_body(in1_ref, in2_ref, out_ref):
    out_ref[:, pl.ds(0, D)] = in1_ref[...].T
    out_ref[:, pl.ds(D, D)] = in2_ref[...].T


def _transpose_split(w):
    return pl.pallas_call(
        _pack_body,
        grid=(NBLK,),
        in_specs=[pl.BlockSpec((D, PACK_BR), lambda i: (0, i)),
                  pl.BlockSpec((D, PACK_BR),
                               lambda i: (0, jnp.minimum(i + NBLK,
                                                         LASTBLK)))],
        out_specs=pl.BlockSpec((PACK_BR, 2 * D), lambda i: (i, 0)),
        out_shape=jax.ShapeDtypeStruct((S, 2 * D), jnp.float32),
    )(w.T, w.T)


def _loss_body(uo_ref, uk_ref, out_ref):
    a = uo_ref[...].reshape(B // 8, 8, L).sum(axis=-1)
    b = uk_ref[...].reshape(B // 8, 8, L).sum(axis=-1)

    def logsig(x):
        return jnp.minimum(x, 0.0) - jnp.log(1.0 + jnp.exp(-jnp.abs(x)))

    out_ref[...] = jnp.full((1, 1), -jnp.sum(logsig(a) + logsig(b)) / B)


def kernel(center, outside, negative, W_center, W_outside):
    center = center.reshape(B)
    outside = outside.reshape(B)
    neg = negative.reshape(B * NEG)
    wop = _transpose_split(W_outside)
    o_lin, s_lin = _sc_kernel_a(outside, neg, wop)
    wcp = _transpose_split(W_center)
    uo, uk = _sc_kernel_c(center, o_lin, s_lin, wcp)
    loss = pl.pallas_call(
        _loss_body,
        out_shape=jax.ShapeDtypeStruct((1, 1), jnp.float32),
    )(uo.reshape(B // 8, 8 * L), uk.reshape(B // 8, 8 * L))
    return loss[0, 0]


# speedup vs baseline: 3.0282x; 1.0415x over previous
import functools

import jax
import jax.numpy as jnp
from jax import lax
from jax.experimental import pallas as pl
from jax.experimental.pallas import tpu as pltpu
from jax.experimental.pallas import tpu_sc as plsc

B = 16384
NEG = 20
D = 64
L = 16
NC = 2
NS = 16
NW = NC * NS
BPW = B // NW

CH_A = 16
NCH_A = BPW // CH_A
NGROW = 80
NGIDX = CH_A * NEG // NGROW

CH_C = 64
NCH_C = BPW // CH_C

PACK_BR = 16384
NBLK = 31
LASTBLK = (1000000 + PACK_BR - 1) // PACK_BR - 1
S = NBLK * PACK_BR


def _sc_a(outside_h, neg_h, wop_h, o_out, s_out,
          idx_o, idx_n, sob, snb, o_rows, n_rows, o_buf, s_buf, sem):
    wid = lax.axis_index("s") * NC + lax.axis_index("c")
    base = wid * BPW
    pltpu.sync_copy(outside_h.at[pl.ds(base, BPW)], idx_o.at[pl.ds(0, BPW)])
    pltpu.sync_copy(neg_h.at[pl.ds(base * NEG, BPW * NEG)], idx_n)

    def chunk_body(t, _):
        for v in range(CH_A // L):
            ov = idx_o[pl.ds(t * CH_A + v * L, L)]
            sob[pl.ds(v * L, L)] = jnp.where(ov >= S, ov - S, ov)
        for v in range(CH_A * NEG // L):
            nv = idx_n[pl.ds(t * CH_A * NEG + v * L, L)]
            snb[pl.ds(v * L, L)] = jnp.where(nv >= S, nv - S, nv)
        cps = [pltpu.async_copy(wop_h.at[sob], o_rows, sem)]
        for j in range(NGIDX):
            cps.append(pltpu.async_copy(
                wop_h.at[snb.at[pl.ds(j * NGROW, NGROW)]],
                n_rows.at[pl.ds(j * NGROW, NGROW)], sem))
        for cp in cps:
            cp.wait()

        def item_body(m, _):
            gi = t * CH_A + m
            offo = jnp.where(idx_o[pl.ds(gi, L)][0] >= S, D, 0)
            nv0 = jnp.where(idx_n[pl.ds(gi * NEG, L)] >= S, D, 0)
            nv1 = jnp.where(idx_n[pl.ds(gi * NEG + NEG - L, L)] >= S, D, 0)
            sj = [n_rows[m * NEG, pl.ds(nv0[0] + j * L, L)]
                  for j in range(D // L)]
            for k in range(1, NEG):
                offn = nv0[k] if k < L else nv1[k - (NEG - L)]
                for j in range(D // L):
                    sj[j] = sj[j] + n_rows[m * NEG + k,
                                           pl.ds(offn + j * L, L)]
            for j in range(D // L):
                o_buf[pl.ds(gi * D + j * L, L)] = o_rows[m, pl.ds(offo
                                                                  + j * L, L)]
                s_buf[pl.ds(gi * D + j * L, L)] = sj[j]
            return 0

        lax.fori_loop(0, CH_A, item_body, 0)
        return 0

    lax.fori_loop(0, NCH_A, chunk_body, 0)
    pltpu.sync_copy(o_buf, o_out.at[pl.ds(base * D, BPW * D)])
    pltpu.sync_copy(s_buf, s_out.at[pl.ds(base * D, BPW * D)])


@functools.partial(
    pl.kernel,
    mesh=plsc.VectorSubcoreMesh(core_axis_name="c", subcore_axis_name="s"),
    out_type=[jax.ShapeDtypeStruct((B * D,), jnp.float32),
              jax.ShapeDtypeStruct((B * D,), jnp.float32)],
    scratch_types=[
        pltpu.VMEM((BPW + L,), jnp.int32),
        pltpu.VMEM((BPW * NEG,), jnp.int32),
        pltpu.VMEM((CH_A,), jnp.int32),
        pltpu.VMEM((CH_A * NEG,), jnp.int32),
        pltpu.VMEM((CH_A, 2 * D), jnp.float32),
        pltpu.VMEM((CH_A * NEG, 2 * D), jnp.float32),
        pltpu.VMEM((BPW * D,), jnp.float32),
        pltpu.VMEM((BPW * D,), jnp.float32),
        pltpu.SemaphoreType.DMA,
    ],
)
def _sc_kernel_a(outside_h, neg_h, wop_h, o_out, s_out,
                 idx_o, idx_n, sob, snb, o_rows, n_rows, o_buf, s_buf, sem):
    _sc_a(outside_h, neg_h, wop_h, o_out, s_out,
          idx_o, idx_n, sob, snb, o_rows, n_rows, o_buf, s_buf, sem)


def _sc_c(center_h, o_h, s_h, wcp_h, uo_out, uk_out,
          idx_c, scb, c_rows, o_l, s_l, uo_buf, uk_buf, sem):
    wid = lax.axis_index("s") * NC + lax.axis_index("c")
    base = wid * BPW
    pltpu.sync_copy(center_h.at[pl.ds(base, BPW)], idx_c.at[pl.ds(0, BPW)])

    def chunk_body(t, _):
        for v in range(CH_C // L):
            iv = idx_c[pl.ds(t * CH_C + v * L, L)]
            scb[pl.ds(v * L, L)] = jnp.where(iv >= S, iv - S, iv)
        cp = pltpu.async_copy(wcp_h.at[scb], c_rows, sem)
        pltpu.sync_copy(o_h.at[pl.ds((base + t * CH_C) * D, CH_C * D)], o_l)
        pltpu.sync_copy(s_h.at[pl.ds((base + t * CH_C) * D, CH_C * D)], s_l)
        cp.wait()

        def item_body(m, _):
            gi = t * CH_C + m
            offc = jnp.where(idx_c[pl.ds(gi, L)][0] >= S, D, 0)
            cj = [c_rows[m, pl.ds(offc + j * L, L)] for j in range(D // L)]
            p = cj[0] * o_l[pl.ds(m * D, L)]
            q = cj[0] * s_l[pl.ds(m * D, L)]
            for j in range(1, D // L):
                p = p + cj[j] * o_l[pl.ds(m * D + j * L, L)]
                q = q + cj[j] * s_l[pl.ds(m * D + j * L, L)]
            uo_buf[pl.ds(gi * L, L)] = p
            uk_buf[pl.ds(gi * L, L)] = -q
            return 0

        lax.fori_loop(0, CH_C, item_body, 0)
        return 0

    lax.fori_loop(0, NCH_C, chunk_body, 0)
    pltpu.sync_copy(uo_buf, uo_out.at[pl.ds(base * L, BPW * L)])
    pltpu.sync_copy(uk_buf, uk_out.at[pl.ds(base * L, BPW * L)])


@functools.partial(
    pl.kernel,
    mesh=plsc.VectorSubcoreMesh(core_axis_name="c", subcore_axis_name="s"),
    out_type=[jax.ShapeDtypeStruct((B * L,), jnp.float32),
              jax.ShapeDtypeStruct((B * L,), jnp.float32)],
    scratch_types=[
        pltpu.VMEM((BPW + L,), jnp.int32),
        pltpu.VMEM((CH_C,), jnp.int32),
        pltpu.VMEM((CH_C, 2 * D), jnp.float32),
        pltpu.VMEM((CH_C * D,), jnp.float32),
        pltpu.VMEM((CH_C * D,), jnp.float32),
        pltpu.VMEM((BPW * L,), jnp.float32),
        pltpu.VMEM((BPW * L,), jnp.float32),
        pltpu.SemaphoreType.DMA,
    ],
)
def _sc_kernel_c(center_h, o_h, s_h, wcp_h, uo_out, uk_out,
                 idx_c, scb, c_rows, o_l, s_l, uo_buf, uk_buf, sem):
    _sc_c(center_h, o_h, s_h, wcp_h, uo_out, uk_out,
          idx_c, scb, c_rows, o_l, s_l, uo_buf, uk_buf, sem)


def _pack_body(in1_ref, in2_ref, out_ref):
    out_ref[:, pl.ds(0, D)] = in1_ref[...].T
    out_ref[:, pl.ds(D, D)] = in2_ref[...].T


def _transpose_split(w):
    return pl.pallas_call(
        _pack_body,
        grid=(NBLK,),
        in_specs=[pl.BlockSpec((D, PACK_BR), lambda i: (0, i)),
                  pl.BlockSpec((D, PACK_BR),
                               lambda i: (0, jnp.minimum(i + NBLK,
                                                         LASTBLK)))],
        out_specs=pl.BlockSpec((PACK_BR, 2 * D), lambda i: (i, 0)),
        out_shape=jax.ShapeDtypeStruct((S, 2 * D), jnp.float32),
    )(w.T, w.T)


def _loss_body(uo_ref, uk_ref, out_ref):
    a = uo_ref[...].reshape(B // 8, 8, L).sum(axis=-1)
    b = uk_ref[...].reshape(B // 8, 8, L).sum(axis=-1)

    def logsig(x):
        return jnp.minimum(x, 0.0) - jnp.log(1.0 + jnp.exp(-jnp.abs(x)))

    out_ref[...] = jnp.full((1, 1), -jnp.sum(logsig(a) + logsig(b)) / B)


def kernel(center, outside, negative, W_center, W_outside):
    center = center.reshape(B)
    outside = outside.reshape(B)
    neg = negative.reshape(B * NEG)
    wop = _transpose_split(W_outside)
    o_lin, s_lin = _sc_kernel_a(outside, neg, wop)
    wcp = _transpose_split(W_center)
    uo, uk = _sc_kernel_c(center, o_lin, s_lin, wcp)
    loss = pl.pallas_call(
        _loss_body,
        out_shape=jax.ShapeDtypeStruct((1, 1), jnp.float32),
    )(uo.reshape(B // 8, 8 * L), uk.reshape(B // 8, 8 * L))
    return loss[0, 0]
